# Initial kernel scaffold; baseline (speedup 1.0000x reference)
import jax
import jax.numpy as jnp
from jax.experimental import pallas as pl


def kernel(q_code, k_code, table):
    raise NotImplementedError("write your pallas kernel here")


# speedup vs baseline: 2662.1261x; 2662.1261x over previous
import functools

import jax
import jax.numpy as jnp
from jax import lax
from jax.experimental import pallas as pl
from jax.experimental.pallas import tpu as pltpu
from jax.experimental.pallas import tpu_sc as plsc

N_SUB = 16
N_CW = 256
Q = 1024
K = 4096
LANES = 16
NUM_WORKERS = 32
Q_PER_W = Q // NUM_WORKERS
K_CHUNKS = K // LANES


def _sc_kernel(table_hbm, qidx_hbm, kidxt_hbm, out_hbm,
               kidx_v, qidx_v, g_v, out_v, gsem):
    wid = lax.axis_index("s") * 2 + lax.axis_index("c")

    pltpu.sync_copy(kidxt_hbm, kidx_v)
    pltpu.sync_copy(qidx_hbm.at[pl.ds(wid * (Q_PER_W * N_SUB), Q_PER_W * N_SUB)],
                    qidx_v)

    def per_q(q_local, _):
        row_ids = qidx_v[pl.ds(q_local * N_SUB, N_SUB)]
        pltpu.async_copy(table_hbm.at[row_ids], g_v, gsem).wait()

        def per_chunk(c, _):
            acc = jnp.zeros((LANES,), jnp.float32)
            for i in range(N_SUB):
                kv = kidx_v[i, pl.ds(c * LANES, LANES)]
                acc = acc + plsc.load_gather(
                    g_v, [jnp.full((LANES,), i, jnp.int32), kv])
            out_v[pl.ds(c * LANES, LANES)] = acc
            return ()

        lax.fori_loop(0, K_CHUNKS, per_chunk, (), unroll=2)
        pltpu.sync_copy(out_v, out_hbm.at[wid * Q_PER_W + q_local])
        return ()

    lax.fori_loop(0, Q_PER_W, per_q, ())


def kernel(q_code, k_code, table):
    table_flat = table.reshape(N_SUB * N_CW, N_CW)
    qidx = (q_code.astype(jnp.int32)
            + jnp.arange(N_SUB, dtype=jnp.int32)[None, :] * N_CW).reshape(-1)
    kidxt = k_code.T.astype(jnp.int32)

    mesh = plsc.VectorSubcoreMesh(core_axis_name="c", subcore_axis_name="s")
    f = functools.partial(
        pl.kernel,
        mesh=mesh,
        compiler_params=pltpu.CompilerParams(use_tc_tiling_on_sc=False,
                                             needs_layout_passes=False),
        out_type=jax.ShapeDtypeStruct((Q, K), jnp.float32),
        scratch_types=[
            pltpu.VMEM((N_SUB, K), jnp.int32),
            pltpu.VMEM((Q_PER_W * N_SUB,), jnp.int32),
            pltpu.VMEM((N_SUB, N_CW), jnp.float32),
            pltpu.VMEM((K,), jnp.float32),
            pltpu.SemaphoreType.DMA,
        ],
    )(_sc_kernel)
    return f(table_flat, qidx, kidxt)


# speedup vs baseline: 3425.2117x; 1.2866x over previous
import functools

import jax
import jax.numpy as jnp
from jax import lax
from jax.experimental import pallas as pl
from jax.experimental.pallas import tpu as pltpu
from jax.experimental.pallas import tpu_sc as plsc

N_SUB = 16
N_CW = 256
Q = 1024
K = 4096
LANES = 16
NUM_WORKERS = 32
Q_PER_W = Q // NUM_WORKERS
QBLK = 8
N_QBLK = Q_PER_W // QBLK
K_HALF = K // 2
CHUNKS_PER_HALF = K_HALF // LANES


def _sc_kernel(table_hbm, qidx_hbm, kidxt_hbm, out_hbm,
               kidx_v, qidx_v, g_v, out_v, gsem):
    wid = lax.axis_index("s") * 2 + lax.axis_index("c")

    pltpu.sync_copy(kidxt_hbm, kidx_v)
    pltpu.sync_copy(qidx_hbm.at[pl.ds(wid * N_QBLK, N_QBLK)], qidx_v)

    def per_qblock(qb, _):
        pltpu.async_copy(table_hbm.at[qidx_v.at[qb]], g_v, gsem).wait()

        for h in range(2):
            def per_chunk(c, _):
                accs = [None] * QBLK
                for i in range(N_SUB):
                    kv = kidx_v[i, pl.ds(h * K_HALF + c * LANES, LANES)]
                    for b in range(QBLK):
                        g = plsc.load_gather(
                            g_v, [jnp.full((LANES,), b * N_SUB + i, jnp.int32),
                                  kv])
                        accs[b] = g if i == 0 else accs[b] + g
                for b in range(QBLK):
                    out_v[b, pl.ds(c * LANES, LANES)] = accs[b]
                return ()

            lax.fori_loop(0, CHUNKS_PER_HALF, per_chunk, ())
            pltpu.sync_copy(
                out_v,
                out_hbm.at[pl.ds((wid * N_QBLK + qb) * QBLK, QBLK),
                           pl.ds(h * K_HALF, K_HALF)])
        return ()

    lax.fori_loop(0, N_QBLK, per_qblock, ())


def kernel(q_code, k_code, table):
    table_flat = table.reshape(N_SUB * N_CW, N_CW)
    qidx = (q_code.astype(jnp.int32)
            + jnp.arange(N_SUB, dtype=jnp.int32)[None, :] * N_CW)
    qidx = qidx.reshape(NUM_WORKERS * N_QBLK, QBLK * N_SUB)
    kidxt = k_code.T.astype(jnp.int32)

    mesh = plsc.VectorSubcoreMesh(core_axis_name="c", subcore_axis_name="s")
    f = functools.partial(
        pl.kernel,
        mesh=mesh,
        compiler_params=pltpu.CompilerParams(use_tc_tiling_on_sc=False,
                                             needs_layout_passes=False),
        out_type=jax.ShapeDtypeStruct((Q, K), jnp.float32),
        scratch_types=[
            pltpu.VMEM((N_SUB, K), jnp.int32),
            pltpu.VMEM((N_QBLK, QBLK * N_SUB), jnp.int32),
            pltpu.VMEM((QBLK * N_SUB, N_CW), jnp.float32),
            pltpu.VMEM((QBLK, K_HALF), jnp.float32),
            pltpu.SemaphoreType.DMA,
        ],
    )(_sc_kernel)
    return f(table_flat, qidx, kidxt)
